# baseline (speedup 1.0000x reference)
import functools

import jax
import jax.numpy as jnp
from jax import lax
from jax.experimental import pallas as pl
from jax.experimental.pallas import tpu as pltpu
from jax.experimental.pallas import tpu_sc as plsc

_DIM = 16
_BATCH = 16384
_N_SCORE = 32
_NUM_ITEMS = 100000

_NC, _NS = 2, 16
_NW = _NC * _NS
_B_W = _BATCH // _NW
_CHUNK = 128
_N_CHUNK = _B_W // _CHUNK

_BI = 12800

_SC_PARAMS = pltpu.CompilerParams(
    use_tc_tiling_on_sc=False, needs_layout_passes=False)


def _sc_score_users(user_t, score_ids):
    mesh = plsc.VectorSubcoreMesh(core_axis_name="c", subcore_axis_name="s")

    @functools.partial(
        pl.kernel,
        mesh=mesh,
        compiler_params=_SC_PARAMS,
        out_type=jax.ShapeDtypeStruct((_DIM, _N_SCORE), jnp.float32),
        scratch_types=[
            pltpu.VMEM((_N_SCORE,), jnp.int32),
            pltpu.VMEM((_DIM, _N_SCORE), jnp.float32),
            pltpu.SemaphoreType.DMA,
        ],
    )
    def k(user_hbm, sid_hbm, su_hbm, sid_v, su_v, sem):
        wid = lax.axis_index("s") * _NC + lax.axis_index("c")

        @pl.when(wid == 0)
        def _():
            pltpu.sync_copy(sid_hbm, sid_v)
            copies = [
                pltpu.async_copy(user_hbm.at[d].at[sid_v], su_v.at[d], sem)
                for d in range(_DIM)
            ]
            for cp in copies:
                cp.wait()
            pltpu.sync_copy(su_v, su_hbm)

    return k(user_t, score_ids)


def _sc_distances(user_t, item_t, user_ids, pos_ids, neg_ids):
    mesh = plsc.VectorSubcoreMesh(core_axis_name="c", subcore_axis_name="s")

    @functools.partial(
        pl.kernel,
        mesh=mesh,
        compiler_params=_SC_PARAMS,
        out_type=[
            jax.ShapeDtypeStruct((_BATCH,), jnp.float32),
            jax.ShapeDtypeStruct((_BATCH,), jnp.float32),
        ],
        scratch_types=[
            pltpu.VMEM((_N_CHUNK, _CHUNK), jnp.int32),
            pltpu.VMEM((_N_CHUNK, _CHUNK), jnp.int32),
            pltpu.VMEM((_N_CHUNK, _CHUNK), jnp.int32),
            pltpu.VMEM((_DIM, _B_W), jnp.float32),
            pltpu.VMEM((_DIM, _B_W), jnp.float32),
            pltpu.VMEM((_DIM, _B_W), jnp.float32),
            pltpu.VMEM((_B_W,), jnp.float32),
            pltpu.VMEM((_B_W,), jnp.float32),
            pltpu.SemaphoreType.DMA,
        ],
    )
    def k(user_hbm, item_hbm, uid_hbm, pid_hbm, nid_hbm,
          pos_hbm, neg_hbm,
          uid_v, pid_v, nid_v, u_v, p_v, n_v, pos_v, neg_v, sem):
        wid = lax.axis_index("s") * _NC + lax.axis_index("c")
        base = wid * _B_W

        for c in range(_N_CHUNK):
            off = base + c * _CHUNK
            pltpu.sync_copy(uid_hbm.at[pl.ds(off, _CHUNK)], uid_v.at[c])
            pltpu.sync_copy(pid_hbm.at[pl.ds(off, _CHUNK)], pid_v.at[c])
            pltpu.sync_copy(nid_hbm.at[pl.ds(off, _CHUNK)], nid_v.at[c])

        copies = []
        for d in range(_DIM):
            for c in range(_N_CHUNK):
                dst = pl.ds(c * _CHUNK, _CHUNK)
                copies.append(pltpu.async_copy(
                    user_hbm.at[d].at[uid_v.at[c]], u_v.at[d].at[dst], sem))
                copies.append(pltpu.async_copy(
                    item_hbm.at[d].at[pid_v.at[c]], p_v.at[d].at[dst], sem))
                copies.append(pltpu.async_copy(
                    item_hbm.at[d].at[nid_v.at[c]], n_v.at[d].at[dst], sem))
        for cp in copies:
            cp.wait()

        def body(g, carry):
            sl = pl.ds(g * 16, 16)
            accp = jnp.zeros((16,), jnp.float32)
            accn = jnp.zeros((16,), jnp.float32)
            for d in range(_DIM):
                u = u_v[d, sl]
                dp = u - p_v[d, sl]
                dn = u - n_v[d, sl]
                accp = accp + dp * dp
                accn = accn + dn * dn
            pos_v[sl] = accp
            neg_v[sl] = accn
            return carry

        lax.fori_loop(0, _B_W // 16, body, 0, unroll=2)

        pltpu.sync_copy(pos_v, pos_hbm.at[pl.ds(base, _B_W)])
        pltpu.sync_copy(neg_v, neg_hbm.at[pl.ds(base, _B_W)])

    return k(user_t, item_t, user_ids, pos_ids, neg_ids)


def _tc_scores(su_t, item_t):
    def body(su_ref, it_ref, out_ref):
        sut = su_ref[...]
        itb = it_ref[...]
        dots = lax.dot_general(sut, itb, (((0,), (0,)), ((), ())),
                               preferred_element_type=jnp.float32)
        su2 = jnp.sum(sut * sut, axis=0)
        it2 = jnp.sum(itb * itb, axis=0)
        out_ref[...] = 2.0 * dots - su2[:, None] - it2[None, :]

    return pl.pallas_call(
        body,
        grid=(pl.cdiv(_NUM_ITEMS, _BI),),
        in_specs=[
            pl.BlockSpec((_DIM, _N_SCORE), lambda i: (0, 0)),
            pl.BlockSpec((_DIM, _BI), lambda i: (0, i)),
        ],
        out_specs=pl.BlockSpec((_N_SCORE, _BI), lambda i: (0, i)),
        out_shape=jax.ShapeDtypeStruct((_N_SCORE, _NUM_ITEMS), jnp.float32),
    )(su_t, item_t)


def kernel(user_embeddings, item_embeddings, user_ids, pos_item_ids,
           neg_item_ids, score_user_ids):
    user_t = user_embeddings.T
    item_t = item_embeddings.T
    su_t = _sc_score_users(user_t, score_user_ids)
    pos_d, neg_d = _sc_distances(
        user_t, item_t, user_ids, pos_item_ids, neg_item_ids)
    scores = _tc_scores(su_t, item_t)
    return (pos_d, neg_d, scores)


# speedup vs baseline: 12.2077x; 12.2077x over previous
import functools

import jax
import jax.numpy as jnp
from jax import lax
from jax.experimental import pallas as pl
from jax.experimental.pallas import tpu as pltpu
from jax.experimental.pallas import tpu_sc as plsc

_DIM = 16
_BATCH = 16384
_N_SCORE = 32
_NUM_USERS = 1000000
_NUM_ITEMS = 100000

_NC, _NS = 2, 16
_NW = _NC * _NS
_B_W = _BATCH // _NW
_CHUNK = 128
_N_CHUNK = _B_W // _CHUNK

_BI = 12800

_SC_PARAMS = pltpu.CompilerParams(
    use_tc_tiling_on_sc=False, needs_layout_passes=False)


def _tc_split_dims(xt, blk):
    n = xt.shape[1]

    def body(x_ref, *out_refs):
        for d in range(_DIM):
            out_refs[d][...] = x_ref[d, :]

    return pl.pallas_call(
        body,
        grid=(pl.cdiv(n, blk),),
        in_specs=[pl.BlockSpec((_DIM, blk), lambda i: (0, i))],
        out_specs=[pl.BlockSpec((blk,), lambda i: (i,))] * _DIM,
        out_shape=[jax.ShapeDtypeStruct((n,), jnp.float32)] * _DIM,
    )(xt)


def _sc_score_users(user_dims, score_ids):
    mesh = plsc.VectorSubcoreMesh(core_axis_name="c", subcore_axis_name="s")

    @functools.partial(
        pl.kernel,
        mesh=mesh,
        compiler_params=_SC_PARAMS,
        out_type=jax.ShapeDtypeStruct((_DIM, _N_SCORE), jnp.float32),
        scratch_types=[
            pltpu.VMEM((_N_SCORE,), jnp.int32),
            pltpu.VMEM((_DIM, _N_SCORE), jnp.float32),
            pltpu.SemaphoreType.DMA,
        ],
    )
    def k(*refs):
        user_hbm = refs[:_DIM]
        sid_hbm, su_hbm, sid_v, su_v, sem = refs[_DIM:]
        wid = lax.axis_index("s") * _NC + lax.axis_index("c")

        @pl.when(wid == 0)
        def _():
            pltpu.sync_copy(sid_hbm, sid_v)
            copies = [
                pltpu.async_copy(user_hbm[d].at[sid_v], su_v.at[d], sem)
                for d in range(_DIM)
            ]
            for cp in copies:
                cp.wait()
            pltpu.sync_copy(su_v, su_hbm)

    return k(*user_dims, score_ids)


def _sc_distances(user_dims, item_dims, user_ids, pos_ids, neg_ids):
    mesh = plsc.VectorSubcoreMesh(core_axis_name="c", subcore_axis_name="s")

    @functools.partial(
        pl.kernel,
        mesh=mesh,
        compiler_params=_SC_PARAMS,
        out_type=[
            jax.ShapeDtypeStruct((_BATCH,), jnp.float32),
            jax.ShapeDtypeStruct((_BATCH,), jnp.float32),
        ],
        scratch_types=[
            pltpu.VMEM((_N_CHUNK, _CHUNK), jnp.int32),
            pltpu.VMEM((_N_CHUNK, _CHUNK), jnp.int32),
            pltpu.VMEM((_N_CHUNK, _CHUNK), jnp.int32),
            pltpu.VMEM((_DIM, _B_W), jnp.float32),
            pltpu.VMEM((_DIM, _B_W), jnp.float32),
            pltpu.VMEM((_DIM, _B_W), jnp.float32),
            pltpu.VMEM((_B_W,), jnp.float32),
            pltpu.VMEM((_B_W,), jnp.float32),
            pltpu.SemaphoreType.DMA,
        ],
    )
    def k(*refs):
        user_hbm = refs[:_DIM]
        item_hbm = refs[_DIM:2 * _DIM]
        (uid_hbm, pid_hbm, nid_hbm, pos_hbm, neg_hbm,
         uid_v, pid_v, nid_v, u_v, p_v, n_v, pos_v, neg_v, sem) = refs[2 * _DIM:]
        wid = lax.axis_index("s") * _NC + lax.axis_index("c")
        base = wid * _B_W

        for c in range(_N_CHUNK):
            off = base + c * _CHUNK
            pltpu.sync_copy(uid_hbm.at[pl.ds(off, _CHUNK)], uid_v.at[c])
            pltpu.sync_copy(pid_hbm.at[pl.ds(off, _CHUNK)], pid_v.at[c])
            pltpu.sync_copy(nid_hbm.at[pl.ds(off, _CHUNK)], nid_v.at[c])

        copies = []
        for d in range(_DIM):
            for c in range(_N_CHUNK):
                dst = pl.ds(c * _CHUNK, _CHUNK)
                copies.append(pltpu.async_copy(
                    user_hbm[d].at[uid_v.at[c]], u_v.at[d].at[dst], sem))
                copies.append(pltpu.async_copy(
                    item_hbm[d].at[pid_v.at[c]], p_v.at[d].at[dst], sem))
                copies.append(pltpu.async_copy(
                    item_hbm[d].at[nid_v.at[c]], n_v.at[d].at[dst], sem))
        for cp in copies:
            cp.wait()

        def body(g, carry):
            sl = pl.ds(g * 16, 16)
            accp = jnp.zeros((16,), jnp.float32)
            accn = jnp.zeros((16,), jnp.float32)
            for d in range(_DIM):
                u = u_v[d, sl]
                dp = u - p_v[d, sl]
                dn = u - n_v[d, sl]
                accp = accp + dp * dp
                accn = accn + dn * dn
            pos_v[sl] = accp
            neg_v[sl] = accn
            return carry

        lax.fori_loop(0, _B_W // 16, body, 0, unroll=2)

        pltpu.sync_copy(pos_v, pos_hbm.at[pl.ds(base, _B_W)])
        pltpu.sync_copy(neg_v, neg_hbm.at[pl.ds(base, _B_W)])

    return k(*user_dims, *item_dims, user_ids, pos_ids, neg_ids)


def _tc_scores(su_t, item_t):
    def body(su_ref, it_ref, out_ref):
        sut = su_ref[...]
        itb = it_ref[...]
        dots = lax.dot_general(sut, itb, (((0,), (0,)), ((), ())),
                               preferred_element_type=jnp.float32)
        su2 = jnp.sum(sut * sut, axis=0)
        it2 = jnp.sum(itb * itb, axis=0)
        out_ref[...] = 2.0 * dots - su2[:, None] - it2[None, :]

    return pl.pallas_call(
        body,
        grid=(pl.cdiv(_NUM_ITEMS, _BI),),
        in_specs=[
            pl.BlockSpec((_DIM, _N_SCORE), lambda i: (0, 0)),
            pl.BlockSpec((_DIM, _BI), lambda i: (0, i)),
        ],
        out_specs=pl.BlockSpec((_N_SCORE, _BI), lambda i: (0, i)),
        out_shape=jax.ShapeDtypeStruct((_N_SCORE, _NUM_ITEMS), jnp.float32),
    )(su_t, item_t)


def kernel(user_embeddings, item_embeddings, user_ids, pos_item_ids,
           neg_item_ids, score_user_ids):
    user_t = user_embeddings.T
    item_t = item_embeddings.T
    user_dims = _tc_split_dims(user_t, 65536)
    item_dims = _tc_split_dims(item_t, 20480)
    su_t = _sc_score_users(user_dims, score_user_ids)
    pos_d, neg_d = _sc_distances(
        user_dims, item_dims, user_ids, pos_item_ids, neg_item_ids)
    scores = _tc_scores(su_t, item_t)
    return (pos_d, neg_d, scores)
